# baseline (speedup 1.0000x reference)
import jax
import jax.numpy as jnp
from jax.experimental import pallas as pl

_BLK = 256


def _copy_zero_kernel(hid_ref, out_ref, attn_ref):
    out_ref[...] = hid_ref[...]
    attn_ref[...] = jnp.zeros_like(attn_ref)


def kernel(hidden_states):
    batch, seq, hid = hidden_states.shape
    h2 = hidden_states.reshape(seq, hid)
    out, attn = pl.pallas_call(
        _copy_zero_kernel,
        grid=(seq // _BLK,),
        in_specs=[pl.BlockSpec((_BLK, hid), lambda i: (i, 0))],
        out_specs=[
            pl.BlockSpec((_BLK, hid), lambda i: (i, 0)),
            pl.BlockSpec((_BLK, seq), lambda i: (i, 0)),
        ],
        out_shape=[
            jax.ShapeDtypeStruct((seq, hid), hidden_states.dtype),
            jax.ShapeDtypeStruct((seq, seq), hidden_states.dtype),
        ],
    )(h2)
    return (out.reshape(batch, seq, hid), attn.reshape(batch, seq, seq))


# speedup vs baseline: 1.0590x; 1.0590x over previous
import jax
import jax.numpy as jnp
from jax.experimental import pallas as pl

_RBLK = 2048
_CSPL = 2


def _copy_zero_kernel(hid_ref, out_ref, attn_ref):
    out_ref[...] = hid_ref[...]
    attn_ref[...] = jnp.zeros_like(attn_ref)


def kernel(hidden_states):
    batch, seq, hid = hidden_states.shape
    h2 = hidden_states.reshape(seq, hid)
    out, attn = pl.pallas_call(
        _copy_zero_kernel,
        grid=(seq // _RBLK, _CSPL),
        in_specs=[pl.BlockSpec((_RBLK, hid // _CSPL), lambda i, j: (i, j))],
        out_specs=[
            pl.BlockSpec((_RBLK, hid // _CSPL), lambda i, j: (i, j)),
            pl.BlockSpec((_RBLK, seq // _CSPL), lambda i, j: (i, j)),
        ],
        out_shape=[
            jax.ShapeDtypeStruct((seq, hid), hidden_states.dtype),
            jax.ShapeDtypeStruct((seq, seq), hidden_states.dtype),
        ],
    )(h2)
    return (out.reshape(batch, seq, hid), attn.reshape(batch, seq, seq))


# speedup vs baseline: 1.0676x; 1.0080x over previous
import jax
import jax.numpy as jnp
from jax.experimental import pallas as pl

_BLK = 1024


def _copy_zero_kernel(hid_ref, out_ref, attn_ref):
    out_ref[...] = hid_ref[...]

    @pl.when(pl.program_id(0) < 2)
    def _fill():
        attn_ref[...] = jnp.zeros_like(attn_ref)


def kernel(hidden_states):
    batch, seq, hid = hidden_states.shape
    h2 = hidden_states.reshape(seq, hid)
    out, attn = pl.pallas_call(
        _copy_zero_kernel,
        grid=(seq // _BLK,),
        in_specs=[pl.BlockSpec((_BLK, hid), lambda i: (i, 0))],
        out_specs=[
            pl.BlockSpec((_BLK, hid), lambda i: (i, 0)),
            pl.BlockSpec((_BLK, seq), lambda i: (i, 0)),
        ],
        out_shape=[
            jax.ShapeDtypeStruct((seq, hid), hidden_states.dtype),
            jax.ShapeDtypeStruct((seq, seq), hidden_states.dtype),
        ],
    )(h2)
    return (out.reshape(batch, seq, hid), attn.reshape(batch, seq, seq))


# speedup vs baseline: 1.0736x; 1.0057x over previous
import jax
import jax.numpy as jnp
from jax.experimental import pallas as pl

_BLK = 1024


def _copy_zero_kernel(hid_ref, out_ref, attn_ref):
    out_ref[...] = hid_ref[...]
    attn_ref[...] = jnp.zeros_like(attn_ref)


def kernel(hidden_states):
    batch, seq, hid = hidden_states.shape
    h2 = hidden_states.reshape(seq, hid)
    out, attn = pl.pallas_call(
        _copy_zero_kernel,
        grid=(seq // _BLK,),
        in_specs=[pl.BlockSpec((_BLK, hid), lambda i: (i, 0))],
        out_specs=[
            pl.BlockSpec((_BLK, hid), lambda i: (i, 0)),
            pl.BlockSpec((_BLK, seq), lambda i: (i, 0)),
        ],
        out_shape=[
            jax.ShapeDtypeStruct((seq, hid), hidden_states.dtype),
            jax.ShapeDtypeStruct((seq, seq), hidden_states.dtype),
        ],
    )(h2)
    return (out.reshape(batch, seq, hid), attn.reshape(batch, seq, seq))
